# Initial kernel scaffold; baseline (speedup 1.0000x reference)
import jax
import jax.numpy as jnp
from jax.experimental import pallas as pl


def kernel(text_features, projected_layer_features, W1_w, W1_b, Wc_w, Wc_b, Wi_w, Wi_b, Wf_w, Wf_b, bc, bi, bf, Wq, Wk, ln_w, ln_b):
    raise NotImplementedError("write your pallas kernel here")


# speedup vs baseline: 1.4796x; 1.4796x over previous
import functools

import jax
import jax.numpy as jnp
from jax import lax
from jax.experimental import pallas as pl
from jax.experimental.pallas import tpu as pltpu

D = 2048
R = D // 4
L = 24
N = 576
T = 128
FINAL_K = 64

_PREC = lax.Precision.DEFAULT


def _dotT(a, w, precision=_PREC):
    return lax.dot_general(a.astype(jnp.bfloat16), w.astype(jnp.bfloat16),
                           (((1,), (1,)), ((), ())),
                           preferred_element_type=jnp.float32,
                           precision=precision)


def _pool_body(x_ref, y_ref):
    y_ref[0] = jnp.mean(x_ref[0], axis=0, keepdims=True)


def _recurrence_body(text_ref, y_ref, w1_ref, w1b_ref, wc_ref, wcb_ref,
                     wi_ref, wib_ref, wf_ref, wfb_ref, wq_ref, lnw_ref,
                     lnb_ref, q_ref):
    tmean = jnp.mean(text_ref[...], axis=0, keepdims=True)
    mu = jnp.mean(tmean, axis=1, keepdims=True)
    var = jnp.mean((tmean - mu) ** 2, axis=1, keepdims=True)
    tg = (tmean - mu) / jnp.sqrt(var + 1e-5)

    def step(l, c):
        yl = y_ref[pl.ds(l, 1), :]
        cn = jax.nn.sigmoid(c)
        comb = jnp.concatenate([cn, yl, tg], axis=1)
        s = jax.nn.relu(_dotT(comb, w1_ref[...]) + w1b_ref[...])
        ct = jnp.tanh(_dotT(s, wc_ref[...]) + wcb_ref[...])
        gi = jax.nn.sigmoid(_dotT(s, wi_ref[...]) + wib_ref[...])
        gf = jax.nn.sigmoid(_dotT(s, wf_ref[...]) + wfb_ref[...])
        return gf * c + gi * ct

    c = lax.fori_loop(0, L - 1, step, jnp.zeros((1, D), jnp.float32))
    qpre = _dotT(c, wq_ref[...])
    mu = jnp.mean(qpre, axis=1, keepdims=True)
    var = jnp.mean((qpre - mu) ** 2, axis=1, keepdims=True)
    q_ref[...] = ((qpre - mu) / jnp.sqrt(var + 1e-5)) * lnw_ref[...] + lnb_ref[...]


def _score_body(x_ref, wk_ref, wqc_ref, sb_ref, z_ref):
    x = x_ref[0].astype(jnp.bfloat16)
    wk = wk_ref[...].astype(jnp.bfloat16)
    vT = lax.dot_general(wk, x, (((1,), (1,)), ((), ())),
                         preferred_element_type=jnp.float32,
                         precision=_PREC)
    wqc = wqc_ref[...]
    dot = jnp.sum(vT * wqc, axis=0, keepdims=True)
    sumv = jnp.sum(vT, axis=0, keepdims=True)
    ssq = jnp.sum(vT * vT, axis=0, keepdims=True)
    mu = sumv * (1.0 / D)
    var = ssq * (1.0 / D) - mu * mu
    denom = jnp.sqrt(var + 1e-5)
    s_sum = sb_ref[0]
    bq = sb_ref[1]
    score = (dot - mu * s_sum) / denom + bq
    m = jnp.mean(score)
    sd = jnp.sqrt(jnp.mean((score - m) ** 2))
    z_ref[0] = (score - m) / (sd + 1e-6)


def kernel(text_features, projected_layer_features, W1_w, W1_b, Wc_w, Wc_b,
           Wi_w, Wi_b, Wf_w, Wf_b, bc, bi, bf, Wq, Wk, ln_w, ln_b):
    X = projected_layer_features

    y = pl.pallas_call(
        _pool_body,
        grid=(L,),
        in_specs=[pl.BlockSpec((1, N, D), lambda l: (l, 0, 0))],
        out_specs=pl.BlockSpec((1, 1, D), lambda l: (l, 0, 0)),
        out_shape=jax.ShapeDtypeStruct((L, 1, D), jnp.float32),
    )(X).reshape(L, D)

    row = lambda v: v.reshape(1, -1)
    q = pl.pallas_call(
        _recurrence_body,
        out_shape=jax.ShapeDtypeStruct((1, D), jnp.float32),
    )(text_features, y, W1_w, row(W1_b), Wc_w, row(Wc_b + bc), Wi_w,
      row(Wi_b + bi), Wf_w, row(Wf_b + bf), Wq, row(ln_w), row(ln_b))

    wq = (ln_w * q[0])
    sb = jnp.stack([jnp.sum(wq), jnp.dot(ln_b, q[0])])

    z = pl.pallas_call(
        _score_body,
        grid=(L,),
        in_specs=[
            pl.BlockSpec((1, N, D), lambda l: (l, 0, 0)),
            pl.BlockSpec((D, D), lambda l: (0, 0)),
            pl.BlockSpec((D, 1), lambda l: (0, 0)),
            pl.BlockSpec(memory_space=pltpu.SMEM),
        ],
        out_specs=pl.BlockSpec((1, 1, N), lambda l: (l, 0, 0)),
        out_shape=jax.ShapeDtypeStruct((L, 1, N), jnp.float32),
    )(X, Wk, wq.reshape(D, 1), sb)

    flat = z.reshape(-1)
    _, top_idx = lax.top_k(flat, FINAL_K)
    return jnp.take(X.reshape(-1, D), top_idx, axis=0)
